# baseline (speedup 1.0000x reference)
import functools

import jax
import jax.numpy as jnp
from jax.experimental import pallas as pl
from jax.experimental.pallas import tpu as pltpu

_B, _T, _J, _C = 64, 128, 21, 3
_NG = _B * _T
_G = 256
_STEPS = _NG // _G
_TPB = _G // _T

_PREDS = {0: [4, 8, 12, 16, 20]}
for _d in range(1, 21):
    _PREDS[_d] = [0] if _d % 4 == 1 else [_d - 1]
_SRCS = {d: _PREDS[d] + [d] for d in range(21)}


def _leaky(x):
    return jnp.where(x > 0, x, 0.2 * x)


def _edge_coefs(asrc, adst):
    out = []
    for d in range(21):
        srcs = _SRCS[d]
        logits = [_leaky(asrc[s] + adst[d]) for s in srcs]
        m = functools.reduce(jnp.maximum, logits)
        es = [jnp.exp(l - m) for l in logits]
        inv = 1.0 / (functools.reduce(lambda a, b: a + b, es) + 1e-16)
        out.append((srcs, [e * inv for e in es]))
    return out


def _body(x_ref, wbd_ref, w1_ref, a1_ref, w2_ref, as2_ref, ad2_ref, b2_ref,
          out_ref):
    xt = x_ref[...].T
    xcat = jnp.concatenate([xt[3 * j:3 * j + 3] for j in range(21)],
                           axis=1)

    fold1 = jnp.dot(a1_ref[...], w1_ref[...],
                    preferred_element_type=jnp.float32)
    alpha1 = jnp.dot(fold1, xcat,
                     preferred_element_type=jnp.float32)
    as1 = [alpha1[0:4, j * _G:(j + 1) * _G] for j in range(21)]
    ad1 = [alpha1[4:8, j * _G:(j + 1) * _G] for j in range(21)]
    xs = [xcat[:, j * _G:(j + 1) * _G] for j in range(21)]

    zs = []
    for srcs, coefs in _edge_coefs(as1, ad1):
        zh = []
        for hd in range(4):
            acc = None
            for c, s in zip(coefs, srcs):
                t = c[hd:hd + 1, :] * xs[s]
                acc = t if acc is None else acc + t
            zh.append(acc)
        zs.append(jnp.concatenate(zh, axis=0))
    zcat = jnp.concatenate(zs, axis=1)
    ones = jnp.ones((1, 21 * _G), dtype=jnp.float32)
    zaug = jnp.concatenate([zcat, ones], axis=0)
    x1cat = jnp.maximum(
        jnp.dot(wbd_ref[...], zaug, preferred_element_type=jnp.float32),
        0.0)

    w2t = w2_ref[...]
    fold2s = jnp.dot(as2_ref[...], w2t, preferred_element_type=jnp.float32)
    fold2d = jnp.dot(ad2_ref[...], w2t, preferred_element_type=jnp.float32)
    waug2 = jnp.concatenate([w2t, fold2s, fold2d], axis=0)
    h2all = jnp.dot(waug2, x1cat, preferred_element_type=jnp.float32)

    h2 = [h2all[0:128, j * _G:(j + 1) * _G] for j in range(21)]
    as2 = [h2all[128:129, j * _G:(j + 1) * _G] for j in range(21)]
    ad2 = [h2all[129:130, j * _G:(j + 1) * _G] for j in range(21)]
    b2c = b2_ref[...]

    acc = None
    for srcs, coefs in _edge_coefs(as2, ad2):
        o = None
        for c, s in zip(coefs, srcs):
            t = c * h2[s]
            o = t if o is None else o + t
        r = jnp.maximum(o + b2c, 0.0)
        acc = r if acc is None else acc + r

    scale = 1.0 / (_J * _T)
    ones_t = jnp.ones((1, _T), dtype=jnp.float32)
    rows = [jax.lax.dot_general(
                ones_t, acc[:, k * _T:(k + 1) * _T],
                (((1,), (1,)), ((), ())),
                preferred_element_type=jnp.float32) * scale
            for k in range(_TPB)]
    out_ref[...] = jnp.concatenate(rows, axis=0)[None]


def kernel(keypoints, W1, a_src1, a_dst1, b1, W2, a_src2, a_dst2, b2):
    xnat = keypoints.reshape(_NG, _J * _C)

    w1t = W1.T
    eye4 = jnp.eye(4, dtype=jnp.float32)
    a1s = (eye4[:, :, None] * a_src1[0][:, None, :]).reshape(4, 256)
    a1d = (eye4[:, :, None] * a_dst1[0][:, None, :]).reshape(4, 256)
    a1t = jnp.concatenate([a1s, a1d], axis=0)
    w1r = w1t.reshape(4, 64, _C)
    wbd = (w1r[:, :, None, :] * eye4[:, None, :, None]).reshape(256, 12)
    wbd = jnp.concatenate([wbd, b1.reshape(256, 1)], axis=1)

    out = pl.pallas_call(
        _body,
        grid=(_STEPS,),
        in_specs=[
            pl.BlockSpec((_G, _J * _C), lambda i: (i, 0)),
            pl.BlockSpec((256, 13), lambda i: (0, 0)),
            pl.BlockSpec((256, _C), lambda i: (0, 0)),
            pl.BlockSpec((8, 256), lambda i: (0, 0)),
            pl.BlockSpec((128, 256), lambda i: (0, 0)),
            pl.BlockSpec((1, 128), lambda i: (0, 0)),
            pl.BlockSpec((1, 128), lambda i: (0, 0)),
            pl.BlockSpec((128, 1), lambda i: (0, 0)),
        ],
        out_specs=pl.BlockSpec((1, _TPB, 128), lambda i: (i, 0, 0)),
        out_shape=jax.ShapeDtypeStruct((_STEPS, _TPB, 128), jnp.float32),
        compiler_params=pltpu.CompilerParams(
            dimension_semantics=("parallel",)),
    )(xnat, wbd, w1t, a1t, W2.T,
      a_src2.reshape(1, 128), a_dst2.reshape(1, 128), b2.reshape(128, 1))

    return out.reshape(_B, 128)


# speedup vs baseline: 1.1601x; 1.1601x over previous
import functools

import jax
import jax.numpy as jnp
from jax.experimental import pallas as pl
from jax.experimental.pallas import tpu as pltpu

_B, _T, _J, _C = 64, 128, 21, 3
_NG = _B * _T
_G = 1024
_STEPS = _NG // _G
_TPB = _G // _T

_PREDS = {0: [4, 8, 12, 16, 20]}
for _d in range(1, 21):
    _PREDS[_d] = [0] if _d % 4 == 1 else [_d - 1]
_SRCS = {d: _PREDS[d] + [d] for d in range(21)}


def _leaky(x):
    return jnp.where(x > 0, x, 0.2 * x)


def _edge_coefs(asrc, adst):
    out = []
    for d in range(21):
        srcs = _SRCS[d]
        logits = [_leaky(asrc[s] + adst[d]) for s in srcs]
        m = functools.reduce(jnp.maximum, logits)
        es = [jnp.exp(l - m) for l in logits]
        inv = 1.0 / (functools.reduce(lambda a, b: a + b, es) + 1e-16)
        out.append((srcs, [e * inv for e in es]))
    return out


def _body(x_ref, wbd_ref, w1_ref, a1_ref, w2_ref, as2_ref, ad2_ref, b2_ref,
          out_ref):
    xt = x_ref[...].T
    xcat = jnp.concatenate([xt[3 * j:3 * j + 3] for j in range(21)],
                           axis=1)

    fold1 = jnp.dot(a1_ref[...], w1_ref[...],
                    preferred_element_type=jnp.float32)
    alpha1 = jnp.dot(fold1, xcat,
                     preferred_element_type=jnp.float32)
    as1 = [alpha1[0:4, j * _G:(j + 1) * _G] for j in range(21)]
    ad1 = [alpha1[4:8, j * _G:(j + 1) * _G] for j in range(21)]
    xs = [xcat[:, j * _G:(j + 1) * _G] for j in range(21)]

    zs = []
    for srcs, coefs in _edge_coefs(as1, ad1):
        zh = []
        for hd in range(4):
            acc = None
            for c, s in zip(coefs, srcs):
                t = c[hd:hd + 1, :] * xs[s]
                acc = t if acc is None else acc + t
            zh.append(acc)
        zs.append(jnp.concatenate(zh, axis=0))
    zcat = jnp.concatenate(zs, axis=1)
    ones = jnp.ones((1, 21 * _G), dtype=jnp.float32)
    zaug = jnp.concatenate([zcat, ones], axis=0)
    x1cat = jnp.maximum(
        jnp.dot(wbd_ref[...], zaug, preferred_element_type=jnp.float32),
        0.0)

    w2t = w2_ref[...]
    fold2s = jnp.dot(as2_ref[...], w2t, preferred_element_type=jnp.float32)
    fold2d = jnp.dot(ad2_ref[...], w2t, preferred_element_type=jnp.float32)
    waug2 = jnp.concatenate([w2t, fold2s, fold2d], axis=0)
    h2all = jnp.dot(waug2, x1cat, preferred_element_type=jnp.float32)

    h2 = [h2all[0:128, j * _G:(j + 1) * _G] for j in range(21)]
    as2 = [h2all[128:129, j * _G:(j + 1) * _G] for j in range(21)]
    ad2 = [h2all[129:130, j * _G:(j + 1) * _G] for j in range(21)]
    b2c = b2_ref[...]

    acc = None
    for srcs, coefs in _edge_coefs(as2, ad2):
        o = None
        for c, s in zip(coefs, srcs):
            t = c * h2[s]
            o = t if o is None else o + t
        r = jnp.maximum(o + b2c, 0.0)
        acc = r if acc is None else acc + r

    scale = 1.0 / (_J * _T)
    ones_t = jnp.ones((1, _T), dtype=jnp.float32)
    rows = [jax.lax.dot_general(
                ones_t, acc[:, k * _T:(k + 1) * _T],
                (((1,), (1,)), ((), ())),
                preferred_element_type=jnp.float32) * scale
            for k in range(_TPB)]
    out_ref[...] = jnp.concatenate(rows, axis=0)[None]


def kernel(keypoints, W1, a_src1, a_dst1, b1, W2, a_src2, a_dst2, b2):
    xnat = keypoints.reshape(_NG, _J * _C)

    w1t = W1.T
    eye4 = jnp.eye(4, dtype=jnp.float32)
    a1s = (eye4[:, :, None] * a_src1[0][:, None, :]).reshape(4, 256)
    a1d = (eye4[:, :, None] * a_dst1[0][:, None, :]).reshape(4, 256)
    a1t = jnp.concatenate([a1s, a1d], axis=0)
    w1r = w1t.reshape(4, 64, _C)
    wbd = (w1r[:, :, None, :] * eye4[:, None, :, None]).reshape(256, 12)
    wbd = jnp.concatenate([wbd, b1.reshape(256, 1)], axis=1)

    out = pl.pallas_call(
        _body,
        grid=(_STEPS,),
        in_specs=[
            pl.BlockSpec((_G, _J * _C), lambda i: (i, 0)),
            pl.BlockSpec((256, 13), lambda i: (0, 0)),
            pl.BlockSpec((256, _C), lambda i: (0, 0)),
            pl.BlockSpec((8, 256), lambda i: (0, 0)),
            pl.BlockSpec((128, 256), lambda i: (0, 0)),
            pl.BlockSpec((1, 128), lambda i: (0, 0)),
            pl.BlockSpec((1, 128), lambda i: (0, 0)),
            pl.BlockSpec((128, 1), lambda i: (0, 0)),
        ],
        out_specs=pl.BlockSpec((1, _TPB, 128), lambda i: (i, 0, 0)),
        out_shape=jax.ShapeDtypeStruct((_STEPS, _TPB, 128), jnp.float32),
        compiler_params=pltpu.CompilerParams(
            dimension_semantics=("parallel",)),
    )(xnat, wbd, w1t, a1t, W2.T,
      a_src2.reshape(1, 128), a_dst2.reshape(1, 128), b2.reshape(128, 1))

    return out.reshape(_B, 128)


# speedup vs baseline: 1.1898x; 1.0256x over previous
import functools

import jax
import jax.numpy as jnp
from jax.experimental import pallas as pl
from jax.experimental.pallas import tpu as pltpu

_B, _T, _J, _C = 64, 128, 21, 3
_NG = _B * _T
_G = 2048
_STEPS = _NG // _G
_TPB = _G // _T

_PREDS = {0: [4, 8, 12, 16, 20]}
for _d in range(1, 21):
    _PREDS[_d] = [0] if _d % 4 == 1 else [_d - 1]
_SRCS = {d: _PREDS[d] + [d] for d in range(21)}


def _leaky(x):
    return jnp.where(x > 0, x, 0.2 * x)


def _edge_coefs(asrc, adst):
    out = []
    for d in range(21):
        srcs = _SRCS[d]
        logits = [_leaky(asrc[s] + adst[d]) for s in srcs]
        m = functools.reduce(jnp.maximum, logits)
        es = [jnp.exp(l - m) for l in logits]
        inv = 1.0 / (functools.reduce(lambda a, b: a + b, es) + 1e-16)
        out.append((srcs, [e * inv for e in es]))
    return out


def _body(x_ref, wbd_ref, w1_ref, a1_ref, w2_ref, as2_ref, ad2_ref, b2_ref,
          out_ref):
    xt = x_ref[...].T
    xcat = jnp.concatenate([xt[3 * j:3 * j + 3] for j in range(21)],
                           axis=1)

    fold1 = jnp.dot(a1_ref[...], w1_ref[...],
                    preferred_element_type=jnp.float32)
    alpha1 = jnp.dot(fold1, xcat,
                     preferred_element_type=jnp.float32)
    as1 = [alpha1[0:4, j * _G:(j + 1) * _G] for j in range(21)]
    ad1 = [alpha1[4:8, j * _G:(j + 1) * _G] for j in range(21)]
    xs = [xcat[:, j * _G:(j + 1) * _G] for j in range(21)]

    zs = []
    for srcs, coefs in _edge_coefs(as1, ad1):
        zh = []
        for hd in range(4):
            acc = None
            for c, s in zip(coefs, srcs):
                t = c[hd:hd + 1, :] * xs[s]
                acc = t if acc is None else acc + t
            zh.append(acc)
        zs.append(jnp.concatenate(zh, axis=0))
    zcat = jnp.concatenate(zs, axis=1)
    ones = jnp.ones((1, 21 * _G), dtype=jnp.float32)
    zaug = jnp.concatenate([zcat, ones], axis=0)
    x1cat = jnp.maximum(
        jnp.dot(wbd_ref[...], zaug, preferred_element_type=jnp.float32),
        0.0)

    w2t = w2_ref[...]
    fold2s = jnp.dot(as2_ref[...], w2t, preferred_element_type=jnp.float32)
    fold2d = jnp.dot(ad2_ref[...], w2t, preferred_element_type=jnp.float32)
    waug2 = jnp.concatenate([w2t, fold2s, fold2d], axis=0)
    h2all = jnp.dot(waug2, x1cat, preferred_element_type=jnp.float32)

    h2 = [h2all[0:128, j * _G:(j + 1) * _G] for j in range(21)]
    as2 = [h2all[128:129, j * _G:(j + 1) * _G] for j in range(21)]
    ad2 = [h2all[129:130, j * _G:(j + 1) * _G] for j in range(21)]
    b2c = b2_ref[...]

    acc = None
    for srcs, coefs in _edge_coefs(as2, ad2):
        o = None
        for c, s in zip(coefs, srcs):
            t = c * h2[s]
            o = t if o is None else o + t
        r = jnp.maximum(o + b2c, 0.0)
        acc = r if acc is None else acc + r

    scale = 1.0 / (_J * _T)
    ones_t = jnp.ones((1, _T), dtype=jnp.float32)
    rows = [jax.lax.dot_general(
                ones_t, acc[:, k * _T:(k + 1) * _T],
                (((1,), (1,)), ((), ())),
                preferred_element_type=jnp.float32) * scale
            for k in range(_TPB)]
    out_ref[...] = jnp.concatenate(rows, axis=0)[None]


def kernel(keypoints, W1, a_src1, a_dst1, b1, W2, a_src2, a_dst2, b2):
    xnat = keypoints.reshape(_NG, _J * _C)

    w1t = W1.T
    eye4 = jnp.eye(4, dtype=jnp.float32)
    a1s = (eye4[:, :, None] * a_src1[0][:, None, :]).reshape(4, 256)
    a1d = (eye4[:, :, None] * a_dst1[0][:, None, :]).reshape(4, 256)
    a1t = jnp.concatenate([a1s, a1d], axis=0)
    w1r = w1t.reshape(4, 64, _C)
    wbd = (w1r[:, :, None, :] * eye4[:, None, :, None]).reshape(256, 12)
    wbd = jnp.concatenate([wbd, b1.reshape(256, 1)], axis=1)

    out = pl.pallas_call(
        _body,
        grid=(_STEPS,),
        in_specs=[
            pl.BlockSpec((_G, _J * _C), lambda i: (i, 0)),
            pl.BlockSpec((256, 13), lambda i: (0, 0)),
            pl.BlockSpec((256, _C), lambda i: (0, 0)),
            pl.BlockSpec((8, 256), lambda i: (0, 0)),
            pl.BlockSpec((128, 256), lambda i: (0, 0)),
            pl.BlockSpec((1, 128), lambda i: (0, 0)),
            pl.BlockSpec((1, 128), lambda i: (0, 0)),
            pl.BlockSpec((128, 1), lambda i: (0, 0)),
        ],
        out_specs=pl.BlockSpec((1, _TPB, 128), lambda i: (i, 0, 0)),
        out_shape=jax.ShapeDtypeStruct((_STEPS, _TPB, 128), jnp.float32),
        compiler_params=pltpu.CompilerParams(
            dimension_semantics=("parallel",)),
    )(xnat, wbd, w1t, a1t, W2.T,
      a_src2.reshape(1, 128), a_dst2.reshape(1, 128), b2.reshape(128, 1))

    return out.reshape(_B, 128)


# speedup vs baseline: 1.2038x; 1.0117x over previous
import functools

import jax
import jax.numpy as jnp
from jax.experimental import pallas as pl
from jax.experimental.pallas import tpu as pltpu

_B, _T, _J, _C = 64, 128, 21, 3
_NG = _B * _T
_G = 4096
_STEPS = _NG // _G
_TPB = _G // _T

_PREDS = {0: [4, 8, 12, 16, 20]}
for _d in range(1, 21):
    _PREDS[_d] = [0] if _d % 4 == 1 else [_d - 1]
_SRCS = {d: _PREDS[d] + [d] for d in range(21)}


def _leaky(x):
    return jnp.where(x > 0, x, 0.2 * x)


def _edge_coefs(asrc, adst):
    out = []
    for d in range(21):
        srcs = _SRCS[d]
        logits = [_leaky(asrc[s] + adst[d]) for s in srcs]
        m = functools.reduce(jnp.maximum, logits)
        es = [jnp.exp(l - m) for l in logits]
        inv = 1.0 / (functools.reduce(lambda a, b: a + b, es) + 1e-16)
        out.append((srcs, [e * inv for e in es]))
    return out


def _body(x_ref, wbd_ref, w1_ref, a1_ref, w2_ref, as2_ref, ad2_ref, b2_ref,
          out_ref):
    xt = x_ref[...].T
    xcat = jnp.concatenate([xt[3 * j:3 * j + 3] for j in range(21)],
                           axis=1)

    fold1 = jnp.dot(a1_ref[...], w1_ref[...],
                    preferred_element_type=jnp.float32)
    alpha1 = jnp.dot(fold1, xcat,
                     preferred_element_type=jnp.float32)
    as1 = [alpha1[0:4, j * _G:(j + 1) * _G] for j in range(21)]
    ad1 = [alpha1[4:8, j * _G:(j + 1) * _G] for j in range(21)]
    xs = [xcat[:, j * _G:(j + 1) * _G] for j in range(21)]

    zs = []
    for srcs, coefs in _edge_coefs(as1, ad1):
        zh = []
        for hd in range(4):
            acc = None
            for c, s in zip(coefs, srcs):
                t = c[hd:hd + 1, :] * xs[s]
                acc = t if acc is None else acc + t
            zh.append(acc)
        zs.append(jnp.concatenate(zh, axis=0))
    zcat = jnp.concatenate(zs, axis=1)
    ones = jnp.ones((1, 21 * _G), dtype=jnp.float32)
    zaug = jnp.concatenate([zcat, ones], axis=0)
    x1cat = jnp.maximum(
        jnp.dot(wbd_ref[...], zaug, preferred_element_type=jnp.float32),
        0.0)

    w2t = w2_ref[...]
    fold2s = jnp.dot(as2_ref[...], w2t, preferred_element_type=jnp.float32)
    fold2d = jnp.dot(ad2_ref[...], w2t, preferred_element_type=jnp.float32)
    waug2 = jnp.concatenate([w2t, fold2s, fold2d], axis=0)
    h2all = jnp.dot(waug2, x1cat, preferred_element_type=jnp.float32)

    h2 = [h2all[0:128, j * _G:(j + 1) * _G] for j in range(21)]
    as2 = [h2all[128:129, j * _G:(j + 1) * _G] for j in range(21)]
    ad2 = [h2all[129:130, j * _G:(j + 1) * _G] for j in range(21)]
    b2c = b2_ref[...]

    acc = None
    for srcs, coefs in _edge_coefs(as2, ad2):
        o = None
        for c, s in zip(coefs, srcs):
            t = c * h2[s]
            o = t if o is None else o + t
        r = jnp.maximum(o + b2c, 0.0)
        acc = r if acc is None else acc + r

    scale = 1.0 / (_J * _T)
    ones_t = jnp.ones((1, _T), dtype=jnp.float32)
    rows = [jax.lax.dot_general(
                ones_t, acc[:, k * _T:(k + 1) * _T],
                (((1,), (1,)), ((), ())),
                preferred_element_type=jnp.float32) * scale
            for k in range(_TPB)]
    out_ref[...] = jnp.concatenate(rows, axis=0)[None]


def kernel(keypoints, W1, a_src1, a_dst1, b1, W2, a_src2, a_dst2, b2):
    xnat = keypoints.reshape(_NG, _J * _C)

    w1t = W1.T
    eye4 = jnp.eye(4, dtype=jnp.float32)
    a1s = (eye4[:, :, None] * a_src1[0][:, None, :]).reshape(4, 256)
    a1d = (eye4[:, :, None] * a_dst1[0][:, None, :]).reshape(4, 256)
    a1t = jnp.concatenate([a1s, a1d], axis=0)
    w1r = w1t.reshape(4, 64, _C)
    wbd = (w1r[:, :, None, :] * eye4[:, None, :, None]).reshape(256, 12)
    wbd = jnp.concatenate([wbd, b1.reshape(256, 1)], axis=1)

    out = pl.pallas_call(
        _body,
        grid=(_STEPS,),
        in_specs=[
            pl.BlockSpec((_G, _J * _C), lambda i: (i, 0)),
            pl.BlockSpec((256, 13), lambda i: (0, 0)),
            pl.BlockSpec((256, _C), lambda i: (0, 0)),
            pl.BlockSpec((8, 256), lambda i: (0, 0)),
            pl.BlockSpec((128, 256), lambda i: (0, 0)),
            pl.BlockSpec((1, 128), lambda i: (0, 0)),
            pl.BlockSpec((1, 128), lambda i: (0, 0)),
            pl.BlockSpec((128, 1), lambda i: (0, 0)),
        ],
        out_specs=pl.BlockSpec((1, _TPB, 128), lambda i: (i, 0, 0)),
        out_shape=jax.ShapeDtypeStruct((_STEPS, _TPB, 128), jnp.float32),
        compiler_params=pltpu.CompilerParams(
            dimension_semantics=("parallel",)),
    )(xnat, wbd, w1t, a1t, W2.T,
      a_src2.reshape(1, 128), a_dst2.reshape(1, 128), b2.reshape(128, 1))

    return out.reshape(_B, 128)


# speedup vs baseline: 1.2409x; 1.0308x over previous
import functools

import jax
import jax.numpy as jnp
from jax.experimental import pallas as pl
from jax.experimental.pallas import tpu as pltpu

_B, _T, _J, _C = 64, 128, 21, 3
_NG = _B * _T
_G = 4096
_STEPS = _NG // _G
_TPB = _G // _T

_PREDS = {0: [4, 8, 12, 16, 20]}
for _d in range(1, 21):
    _PREDS[_d] = [0] if _d % 4 == 1 else [_d - 1]
_SRCS = {d: _PREDS[d] + [d] for d in range(21)}


def _leaky(x):
    return jnp.maximum(x, 0.2 * x)


def _edge_coefs(asrc, adst):
    out = []
    for d in range(21):
        srcs = _SRCS[d]
        logits = [_leaky(asrc[s] + adst[d]) for s in srcs]
        if len(srcs) == 2:
            cp = 1.0 / (1.0 + jnp.exp(logits[1] - logits[0]))
            out.append((srcs, [cp, 1.0 - cp]))
            continue
        m = functools.reduce(jnp.maximum, logits)
        es = [jnp.exp(l - m) for l in logits]
        inv = 1.0 / (functools.reduce(lambda a, b: a + b, es) + 1e-16)
        out.append((srcs, [e * inv for e in es]))
    return out


def _body(x_ref, wbd_ref, w1_ref, a1_ref, w2_ref, as2_ref, ad2_ref, b2_ref,
          out_ref):
    xt = x_ref[...].T
    xcat = jnp.concatenate([xt[3 * j:3 * j + 3] for j in range(21)],
                           axis=1)

    fold1 = jnp.dot(a1_ref[...], w1_ref[...],
                    preferred_element_type=jnp.float32)
    alpha1 = jnp.dot(fold1, xcat,
                     preferred_element_type=jnp.float32)
    as1 = [alpha1[0:4, j * _G:(j + 1) * _G] for j in range(21)]
    ad1 = [alpha1[4:8, j * _G:(j + 1) * _G] for j in range(21)]
    xs = [xcat[:, j * _G:(j + 1) * _G] for j in range(21)]

    zs = []
    for srcs, coefs in _edge_coefs(as1, ad1):
        zh = []
        for hd in range(4):
            acc = None
            for c, s in zip(coefs, srcs):
                t = c[hd:hd + 1, :] * xs[s]
                acc = t if acc is None else acc + t
            zh.append(acc)
        zs.append(jnp.concatenate(zh, axis=0))
    zcat = jnp.concatenate(zs, axis=1)
    ones = jnp.ones((1, 21 * _G), dtype=jnp.float32)
    zaug = jnp.concatenate([zcat, ones], axis=0)
    x1cat = jnp.maximum(
        jnp.dot(wbd_ref[...], zaug, preferred_element_type=jnp.float32),
        0.0)

    w2t = w2_ref[...]
    fold2s = jnp.dot(as2_ref[...], w2t, preferred_element_type=jnp.float32)
    fold2d = jnp.dot(ad2_ref[...], w2t, preferred_element_type=jnp.float32)
    waug2 = jnp.concatenate([w2t, fold2s, fold2d], axis=0)
    h2all = jnp.dot(waug2, x1cat, preferred_element_type=jnp.float32)

    h2 = [h2all[0:128, j * _G:(j + 1) * _G] for j in range(21)]
    as2 = [h2all[128:129, j * _G:(j + 1) * _G] for j in range(21)]
    ad2 = [h2all[129:130, j * _G:(j + 1) * _G] for j in range(21)]
    b2c = b2_ref[...]

    acc = None
    for srcs, coefs in _edge_coefs(as2, ad2):
        o = None
        for c, s in zip(coefs, srcs):
            t = c * h2[s]
            o = t if o is None else o + t
        r = jnp.maximum(o + b2c, 0.0)
        acc = r if acc is None else acc + r

    scale = 1.0 / (_J * _T)
    ones_t = jnp.ones((1, _T), dtype=jnp.float32)
    rows = [jax.lax.dot_general(
                ones_t, acc[:, k * _T:(k + 1) * _T],
                (((1,), (1,)), ((), ())),
                preferred_element_type=jnp.float32) * scale
            for k in range(_TPB)]
    out_ref[...] = jnp.concatenate(rows, axis=0)[None]


def kernel(keypoints, W1, a_src1, a_dst1, b1, W2, a_src2, a_dst2, b2):
    xnat = keypoints.reshape(_NG, _J * _C)

    w1t = W1.T
    eye4 = jnp.eye(4, dtype=jnp.float32)
    a1s = (eye4[:, :, None] * a_src1[0][:, None, :]).reshape(4, 256)
    a1d = (eye4[:, :, None] * a_dst1[0][:, None, :]).reshape(4, 256)
    a1t = jnp.concatenate([a1s, a1d], axis=0)
    w1r = w1t.reshape(4, 64, _C)
    wbd = (w1r[:, :, None, :] * eye4[:, None, :, None]).reshape(256, 12)
    wbd = jnp.concatenate([wbd, b1.reshape(256, 1)], axis=1)

    out = pl.pallas_call(
        _body,
        grid=(_STEPS,),
        in_specs=[
            pl.BlockSpec((_G, _J * _C), lambda i: (i, 0)),
            pl.BlockSpec((256, 13), lambda i: (0, 0)),
            pl.BlockSpec((256, _C), lambda i: (0, 0)),
            pl.BlockSpec((8, 256), lambda i: (0, 0)),
            pl.BlockSpec((128, 256), lambda i: (0, 0)),
            pl.BlockSpec((1, 128), lambda i: (0, 0)),
            pl.BlockSpec((1, 128), lambda i: (0, 0)),
            pl.BlockSpec((128, 1), lambda i: (0, 0)),
        ],
        out_specs=pl.BlockSpec((1, _TPB, 128), lambda i: (i, 0, 0)),
        out_shape=jax.ShapeDtypeStruct((_STEPS, _TPB, 128), jnp.float32),
        compiler_params=pltpu.CompilerParams(
            dimension_semantics=("parallel",)),
    )(xnat, wbd, w1t, a1t, W2.T,
      a_src2.reshape(1, 128), a_dst2.reshape(1, 128), b2.reshape(128, 1))

    return out.reshape(_B, 128)
